# baseline (speedup 1.0000x reference)
import functools

import jax
import jax.numpy as jnp
from jax import lax
from jax.experimental import pallas as pl
from jax.experimental.pallas import tpu as pltpu
from jax.experimental.pallas import tpu_sc as plsc

_VOCAB = 1000000
_EMB = 64
_B = 4096
_L = 200
_N = _B * _L
_SCALE = 8.0

_NC = 2
_NS = 16
_NW = _NC * _NS
_CH = 128
_GRP = _N // (_NW * _CH)
_DEPTH = 4


def _emb_body(tokens_hbm, table_hbm, out_hbm, idx_v, gbuf, obuf, *sems):
    gsems = sems[:_DEPTH]
    osems = sems[_DEPTH:]

    wid = lax.axis_index("s") * _NC + lax.axis_index("c")
    g0 = wid * _GRP

    pltpu.sync_copy(tokens_hbm.at[pl.ds(g0, _GRP)], idx_v)

    def start_gather(g, b):
        pltpu.async_copy(table_hbm.at[idx_v.at[g]], gbuf.at[b], gsems[b])

    def wait_gather(g, b):
        pltpu.make_async_copy(
            table_hbm.at[idx_v.at[g]], gbuf.at[b], gsems[b]
        ).wait()

    def start_out(g, b):
        pltpu.async_copy(
            obuf.at[b], out_hbm.at[pl.ds((g0 + g) * _CH, _CH)], osems[b]
        )

    def wait_out(g, b):
        pltpu.make_async_copy(
            obuf.at[b], out_hbm.at[pl.ds((g0 + g) * _CH, _CH)], osems[b]
        ).wait()

    for b in range(_DEPTH):
        start_gather(b, b)

    def round_body(i, carry):
        for b in range(_DEPTH):
            g = _DEPTH * i + b
            wait_gather(g, b)

            @pl.when(g >= _DEPTH)
            def _():
                wait_out(g - _DEPTH, b)

            def scale_row(r, c):
                for j in range(_EMB // 16):
                    sl = pl.ds(j * 16, 16)
                    obuf[b, r, sl] = gbuf[b, r, sl] * _SCALE
                return c

            lax.fori_loop(0, _CH, scale_row, 0, unroll=8)

            start_out(g, b)

            @pl.when(g + _DEPTH < _GRP)
            def _():
                start_gather(g + _DEPTH, b)

        return carry

    lax.fori_loop(0, _GRP // _DEPTH, round_body, 0)

    for b in range(_DEPTH):
        wait_out(_GRP - _DEPTH + b, b)


@jax.jit
def _embed(tokens2d, table):
    run = functools.partial(
        pl.kernel,
        mesh=plsc.VectorSubcoreMesh(core_axis_name="c", subcore_axis_name="s"),
        out_type=jax.ShapeDtypeStruct((_N, _EMB), jnp.float32),
        scratch_types=[
            pltpu.VMEM((_GRP, _CH), jnp.int32),
            pltpu.VMEM((_DEPTH, _CH, _EMB), jnp.float32),
            pltpu.VMEM((_DEPTH, _CH, _EMB), jnp.float32),
        ]
        + [pltpu.SemaphoreType.DMA] * (2 * _DEPTH),
        compiler_params=pltpu.CompilerParams(use_tc_tiling_on_sc=False),
    )(_emb_body)
    return run(tokens2d, table)


def kernel(tokens, table):
    tokens2d = tokens.reshape(_N // _CH, _CH)
    out = _embed(tokens2d, table)
    return out.reshape(_B, _L, _EMB)
